# Initial kernel scaffold; baseline (speedup 1.0000x reference)
import jax
import jax.numpy as jnp
from jax.experimental import pallas as pl


def kernel(images, mask, order):
    raise NotImplementedError("write your pallas kernel here")


# speedup vs baseline: 64.1825x; 64.1825x over previous
import functools

import jax
import jax.numpy as jnp
from jax import lax
from jax.experimental import pallas as pl
from jax.experimental.pallas import tpu as pltpu
from jax.experimental.pallas import tpu_sc as plsc

NC = 2
NS = 16
L = 16


def _widx():
    return lax.axis_index("s") * NC + lax.axis_index("c")


@functools.cache
def _resolve_kernel(B, M, N_in, K):
    assert M % L == 0
    zero_row = B * N_in
    mesh = plsc.VectorSubcoreMesh(core_axis_name="c", subcore_axis_name="s")

    @functools.partial(
        pl.kernel,
        out_type=jax.ShapeDtypeStruct((B * M,), jnp.int32),
        mesh=mesh,
        scratch_types=[
            pltpu.VMEM((M,), jnp.int32),
            pltpu.VMEM((2, K), jnp.int32),
            pltpu.VMEM((M,), jnp.int32),
            pltpu.VMEM((M,), jnp.int32),
        ],
        compiler_params=pltpu.CompilerParams(needs_layout_passes=False),
    )
    def resolve(mask_hbm, order_hbm, idx_hbm, mp_v, order_v, g_v, out_v):
        wid = _widx()

        @pl.when(wid < B)
        def _():
            b = wid
            pltpu.sync_copy(mask_hbm.at[b], mp_v)
            pltpu.sync_copy(order_hbm.at[b], order_v)
            boff = b * N_in

            def p1(i, carry):
                v = mp_v[pl.ds(i * L, L)]
                cs = plsc.cumsum(v)
                tot = jnp.sum(v, axis=0)
                posz = jnp.where(v > 0, cs + (carry + boff - 1), zero_row)
                mp_v[pl.ds(i * L, L)] = posz
                g_v[pl.ds(i * L, L)] = lax.iota(jnp.int32, L) + i * L
                return carry + tot

            lax.fori_loop(0, M // L, p1, jnp.int32(0))

            lane0 = lax.iota(jnp.int32, L) == 0
            zeros = jnp.zeros((L,), jnp.int32)
            ones = jnp.ones((L,), jnp.int32)

            def p2(k, _):
                kv = jnp.full((L,), K - 1 - k, jnp.int32)
                fv = plsc.load_gather(order_v, [zeros, kv])
                tv = plsc.load_gather(order_v, [ones, kv])
                gf = plsc.load_gather(g_v, [fv])
                plsc.store_scatter(g_v, [tv], gf, mask=lane0)
                return 0

            lax.fori_loop(0, K, p2, 0)

            def p3(i, _):
                gv = g_v[pl.ds(i * L, L)]
                out_v[pl.ds(i * L, L)] = plsc.load_gather(mp_v, [gv])
                return 0

            lax.fori_loop(0, M // L, p3, 0)
            pltpu.sync_copy(out_v, idx_hbm.at[pl.ds(b * M, M)])

    return resolve


@functools.cache
def _gather_kernel(R, C, n_table):
    CHUNK = 128
    assert R % CHUNK == 0
    n_chunks = R // CHUNK
    n_tiles = NC * NS
    per_tile = -(-n_chunks // n_tiles)
    mesh = plsc.VectorSubcoreMesh(core_axis_name="c", subcore_axis_name="s")

    @functools.partial(
        pl.kernel,
        out_type=jax.ShapeDtypeStruct((R, C), jnp.float32),
        mesh=mesh,
        scratch_types=[
            pltpu.VMEM((CHUNK,), jnp.int32),
            pltpu.VMEM((CHUNK, C), jnp.float32),
            pltpu.SemaphoreType.DMA,
        ],
    )
    def gather(table_hbm, idx_hbm, out_hbm, idx_v, rows_v, sem):
        wid = _widx()

        def chunk(j, _):
            cid = wid + n_tiles * j

            @pl.when(cid < n_chunks)
            def _():
                base = cid * CHUNK
                pltpu.sync_copy(idx_hbm.at[pl.ds(base, CHUNK)], idx_v)
                pltpu.async_copy(table_hbm.at[idx_v], rows_v, sem).wait()
                pltpu.sync_copy(rows_v, out_hbm.at[pl.ds(base, CHUNK)])

            return 0

        lax.fori_loop(0, per_tile, chunk, 0)

    return gather


def kernel(images, mask, order):
    B, N_in, C = images.shape
    M = mask.shape[1]
    K = order.shape[2]

    idx = _resolve_kernel(B, M, N_in, K)(
        mask.astype(jnp.int32), order.astype(jnp.int32)
    )
    table = jnp.concatenate(
        [images.reshape(B * N_in, C), jnp.zeros((8, C), images.dtype)], axis=0
    )
    out = _gather_kernel(B * M, C, B * N_in + 8)(table, idx)
    return out.reshape(B, M, C)


# speedup vs baseline: 66.4013x; 1.0346x over previous
import functools

import jax
import jax.numpy as jnp
from jax import lax
from jax.experimental import pallas as pl
from jax.experimental.pallas import tpu as pltpu
from jax.experimental.pallas import tpu_sc as plsc

NC = 2
NS = 16
L = 16


def _widx():
    return lax.axis_index("s") * NC + lax.axis_index("c")


@functools.cache
def _resolve_kernel(B, M, N_in, K):
    assert M % L == 0
    zero_row = B * N_in
    mesh = plsc.VectorSubcoreMesh(core_axis_name="c", subcore_axis_name="s")

    @functools.partial(
        pl.kernel,
        out_type=jax.ShapeDtypeStruct((B * M,), jnp.int32),
        mesh=mesh,
        scratch_types=[
            pltpu.VMEM((M,), jnp.int32),
            pltpu.VMEM((2, K), jnp.int32),
            pltpu.VMEM((M,), jnp.int32),
            pltpu.VMEM((M,), jnp.int32),
        ],
        compiler_params=pltpu.CompilerParams(needs_layout_passes=False),
    )
    def resolve(mask_hbm, order_hbm, idx_hbm, mp_v, order_v, g_v, out_v):
        wid = _widx()

        @pl.when(wid < B)
        def _():
            b = wid
            pltpu.sync_copy(mask_hbm.at[b], mp_v)
            pltpu.sync_copy(order_hbm.at[b], order_v)
            boff = b * N_in
            iota = lax.iota(jnp.int32, L)

            def lane_bcast(v, j):
                return v.at[jnp.full((L,), j, jnp.int32)].get(
                    mode="promise_in_bounds"
                )

            def p1(i, carry):
                v = mp_v[pl.ds(i * L, L)]
                cs = plsc.cumsum(v)
                posz = jnp.where(v > 0, cs + carry + (boff - 1), zero_row)
                mp_v[pl.ds(i * L, L)] = posz
                g_v[pl.ds(i * L, L)] = iota + i * L
                return carry + lane_bcast(cs, L - 1)

            lax.fori_loop(0, M // L, p1, jnp.zeros((L,), jnp.int32))

            lane0 = iota == 0
            rots = [jnp.where(iota < L - s, iota + s, iota + s - L)
                    for s in range(1, L)]

            def p2(i, _):
                base = K - (i + 1) * L
                fv = lax.rev(order_v[0, pl.ds(base, L)], (0,))
                tv = lax.rev(order_v[1, pl.ds(base, L)], (0,))
                conf = jnp.zeros((L,), jnp.bool_)
                for r in rots:
                    fr = fv.at[r].get(mode="promise_in_bounds")
                    tr = tv.at[r].get(mode="promise_in_bounds")
                    conf = conf | (tv == fr) | (tv == tr)

                def fast():
                    gf = plsc.load_gather(g_v, [fv])
                    plsc.store_scatter(g_v, [tv], gf)

                def slow():
                    for j in range(L):
                        fj = lane_bcast(fv, j)
                        tj = lane_bcast(tv, j)
                        gf = plsc.load_gather(g_v, [fj])
                        plsc.store_scatter(g_v, [tj], gf, mask=lane0)

                lax.cond(jnp.any(conf), slow, fast)
                return 0

            lax.fori_loop(0, K // L, p2, 0)

            def p3(i, _):
                gv = g_v[pl.ds(i * L, L)]
                out_v[pl.ds(i * L, L)] = plsc.load_gather(mp_v, [gv])
                return 0

            lax.fori_loop(0, M // L, p3, 0)
            pltpu.sync_copy(out_v, idx_hbm.at[pl.ds(b * M, M)])

    return resolve


@functools.cache
def _gather_kernel(R, C, n_table):
    CHUNK = 128
    assert R % CHUNK == 0
    n_chunks = R // CHUNK
    n_tiles = NC * NS
    per_tile = -(-n_chunks // n_tiles)
    mesh = plsc.VectorSubcoreMesh(core_axis_name="c", subcore_axis_name="s")

    @functools.partial(
        pl.kernel,
        out_type=jax.ShapeDtypeStruct((R, C), jnp.float32),
        mesh=mesh,
        scratch_types=[
            pltpu.VMEM((CHUNK,), jnp.int32),
            pltpu.VMEM((CHUNK, C), jnp.float32),
            pltpu.SemaphoreType.DMA,
        ],
    )
    def gather(table_hbm, idx_hbm, out_hbm, idx_v, rows_v, sem):
        wid = _widx()

        def chunk(j, _):
            cid = wid + n_tiles * j

            @pl.when(cid < n_chunks)
            def _():
                base = cid * CHUNK
                pltpu.sync_copy(idx_hbm.at[pl.ds(base, CHUNK)], idx_v)
                pltpu.async_copy(table_hbm.at[idx_v], rows_v, sem).wait()
                pltpu.sync_copy(rows_v, out_hbm.at[pl.ds(base, CHUNK)])

            return 0

        lax.fori_loop(0, per_tile, chunk, 0)

    return gather


def kernel(images, mask, order):
    B, N_in, C = images.shape
    M = mask.shape[1]
    K = order.shape[2]

    idx = _resolve_kernel(B, M, N_in, K)(
        mask.astype(jnp.int32), order.astype(jnp.int32)
    )
    table = jnp.concatenate(
        [images.reshape(B * N_in, C), jnp.zeros((8, C), images.dtype)], axis=0
    )
    out = _gather_kernel(B * M, C, B * N_in + 8)(table, idx)
    return out.reshape(B, M, C)


# speedup vs baseline: 795.4787x; 11.9799x over previous
import functools

import jax
import jax.numpy as jnp
from jax import lax
from jax.experimental import pallas as pl
from jax.experimental.pallas import tpu as pltpu
from jax.experimental.pallas import tpu_sc as plsc

NC = 2
NS = 16
L = 16


def _widx():
    return lax.axis_index("s") * NC + lax.axis_index("c")


NPAD = 2048


@functools.cache
def _resolve_kernel(B, M, N_in, K):
    assert M % L == 0
    zero_row = B * N_in
    mesh = plsc.VectorSubcoreMesh(core_axis_name="c", subcore_axis_name="s")

    @functools.partial(
        pl.kernel,
        out_type=jax.ShapeDtypeStruct((B * M,), jnp.int32),
        mesh=mesh,
        scratch_types=[
            pltpu.VMEM((M,), jnp.int32),
            pltpu.VMEM((2, K), jnp.int32),
            pltpu.VMEM((M,), jnp.int32),
            pltpu.VMEM((M,), jnp.int32),
        ],
        compiler_params=pltpu.CompilerParams(needs_layout_passes=False),
    )
    def resolve(mask_hbm, order_hbm, idx_hbm, mp_v, order_v, g_v, out_v):
        wid = _widx()

        @pl.when(wid < B)
        def _():
            b = wid
            pltpu.sync_copy(mask_hbm.at[b], mp_v)
            pltpu.sync_copy(order_hbm.at[b], order_v)
            boff = b * N_in
            iota = lax.iota(jnp.int32, L)

            def lane_bcast(v, j):
                return v.at[jnp.full((L,), j, jnp.int32)].get(
                    mode="promise_in_bounds"
                )

            def p1(i, carry):
                v = mp_v[pl.ds(i * L, L)]
                cs = plsc.cumsum(v)
                zspread = zero_row + ((iota + i * L) & (NPAD - 1))
                posz = jnp.where(v > 0, cs + carry + (boff - 1), zspread)
                mp_v[pl.ds(i * L, L)] = posz
                g_v[pl.ds(i * L, L)] = iota + i * L
                return carry + lane_bcast(cs, L - 1)

            lax.fori_loop(0, M // L, p1, jnp.zeros((L,), jnp.int32))

            lane0 = iota == 0
            rots = [jnp.where(iota < L - s, iota + s, iota + s - L)
                    for s in range(1, L)]

            def p2(i, _):
                base = K - (i + 1) * L
                fv = lax.rev(order_v[0, pl.ds(base, L)], (0,))
                tv = lax.rev(order_v[1, pl.ds(base, L)], (0,))
                conf = jnp.zeros((L,), jnp.bool_)
                for r in rots:
                    fr = fv.at[r].get(mode="promise_in_bounds")
                    tr = tv.at[r].get(mode="promise_in_bounds")
                    conf = conf | (tv == fr) | (tv == tr)

                def fast():
                    gf = plsc.load_gather(g_v, [fv])
                    plsc.store_scatter(g_v, [tv], gf)

                def slow():
                    for j in range(L):
                        fj = lane_bcast(fv, j)
                        tj = lane_bcast(tv, j)
                        gf = plsc.load_gather(g_v, [fj])
                        plsc.store_scatter(g_v, [tj], gf, mask=lane0)

                lax.cond(jnp.any(conf), slow, fast)
                return 0

            lax.fori_loop(0, K // L, p2, 0)

            def p3(i, _):
                gv = g_v[pl.ds(i * L, L)]
                out_v[pl.ds(i * L, L)] = plsc.load_gather(mp_v, [gv])
                return 0

            lax.fori_loop(0, M // L, p3, 0)
            pltpu.sync_copy(out_v, idx_hbm.at[pl.ds(b * M, M)])

    return resolve


@functools.cache
def _gather_kernel(R, C, n_table):
    CHUNK = 128
    assert R % CHUNK == 0
    n_chunks = R // CHUNK
    n_tiles = NC * NS
    per_tile = -(-n_chunks // n_tiles)
    mesh = plsc.VectorSubcoreMesh(core_axis_name="c", subcore_axis_name="s")

    @functools.partial(
        pl.kernel,
        out_type=jax.ShapeDtypeStruct((R, C), jnp.float32),
        mesh=mesh,
        scratch_types=[
            pltpu.VMEM((CHUNK,), jnp.int32),
            pltpu.VMEM((CHUNK,), jnp.int32),
            pltpu.VMEM((CHUNK, C), jnp.float32),
            pltpu.VMEM((CHUNK, C), jnp.float32),
            pltpu.SemaphoreType.DMA,
            pltpu.SemaphoreType.DMA,
            pltpu.SemaphoreType.DMA,
            pltpu.SemaphoreType.DMA,
        ],
    )
    def gather(table_hbm, idx_hbm, out_hbm, i0, i1, r0, r1, sg0, sg1, sw0, sw1):
        wid = _widx()
        idx_v = [i0, i1]
        rows_v = [r0, r1]
        sg = [sg0, sg1]
        sw = [sw0, sw1]

        def cid(j):
            return wid + n_tiles * j

        def gather_desc(s):
            return pltpu.make_async_copy(table_hbm.at[idx_v[s]], rows_v[s], sg[s])

        def wb_desc(s, j):
            return pltpu.make_async_copy(
                rows_v[s], out_hbm.at[pl.ds(cid(j) * CHUNK, CHUNK)], sw[s]
            )

        for j in range(per_tile + 2):
            s = j % 2
            if j >= 2:
                @pl.when(cid(j - 2) < n_chunks)
                def _(j=j, s=s):
                    wb_desc(s, j - 2).wait()
            if j < per_tile:
                @pl.when(cid(j) < n_chunks)
                def _(j=j, s=s):
                    pltpu.sync_copy(idx_hbm.at[cid(j)], idx_v[s])
                    gather_desc(s).start()
            if 1 <= j <= per_tile:
                ps = (j - 1) % 2

                @pl.when(cid(j - 1) < n_chunks)
                def _(j=j, ps=ps):
                    gather_desc(ps).wait()
                    wb_desc(ps, j - 1).start()

    return gather


def kernel(images, mask, order):
    B, N_in, C = images.shape
    M = mask.shape[1]
    K = order.shape[2]

    idx = _resolve_kernel(B, M, N_in, K)(
        mask.astype(jnp.int32), order.astype(jnp.int32)
    )
    table = jnp.concatenate(
        [images.reshape(B * N_in, C), jnp.zeros((NPAD, C), images.dtype)], axis=0
    )
    out = _gather_kernel(B * M, C, B * N_in + NPAD)(
        table, idx.reshape(B * M // 128, 128)
    )
    return out.reshape(B, M, C)
